# Initial kernel scaffold; baseline (speedup 1.0000x reference)
import jax
import jax.numpy as jnp
from jax.experimental import pallas as pl


def kernel(x, edge_index, batch, lin0_W, lin0_b, conv_W, conv_b, W_ih, W_hh, b_ih, b_hh, lin1_W, lin1_b, lin2_W, lin2_b):
    raise NotImplementedError("write your pallas kernel here")


# speedup vs baseline: 9.9093x; 9.9093x over previous
import functools

import jax
import jax.numpy as jnp
from jax import lax
from jax.experimental import pallas as pl
from jax.experimental.pallas import tpu as pltpu
from jax.experimental.pallas import tpu_sc as plsc

N = 50000
E = 800000
MOL_IN = 25
DIM = 64
HD = 32
B = 128
STEPS = 3

NC = 2
NS = 16
CHUNK = 128
NBUF = 4
KCH = 392
E_PAD = NS * CHUNK * KCH
ACC_ROWS = 50048
TROWS = ACC_ROWS // NS
ZROWS = 184

RC = 2000
NBC = N // RC
G_GRID = STEPS * NBC + 1

RA = 2000
NBA = N // RA


def _lin0_body(x_ref, wa_ref, wb_ref, b_ref, lo_ref, hi_ref):
    x = x_ref[...]
    b = b_ref[...]
    lo = lax.dot_general(x, wa_ref[...], (((1,), (1,)), ((), ())),
                         preferred_element_type=jnp.float32,
                         precision=lax.Precision.DEFAULT)
    hi = lax.dot_general(x, wb_ref[...], (((1,), (1,)), ((), ())),
                         preferred_element_type=jnp.float32,
                         precision=lax.Precision.DEFAULT)
    lo_ref[...] = jnp.maximum(lo + b[:, 0:HD], 0.0)
    hi_ref[...] = jnp.maximum(hi + b[:, HD:DIM], 0.0)


def _lin0(x, lin0_W, lin0_b):
    wa = lin0_W[0:HD, :]
    wb = lin0_W[HD:DIM, :]
    b2 = lin0_b.reshape(1, DIM)
    return pl.pallas_call(
        _lin0_body,
        grid=(NBA,),
        in_specs=[
            pl.BlockSpec((RA, MOL_IN), lambda i: (i, 0)),
            pl.BlockSpec((HD, MOL_IN), lambda i: (0, 0)),
            pl.BlockSpec((HD, MOL_IN), lambda i: (0, 0)),
            pl.BlockSpec((1, DIM), lambda i: (0, 0)),
        ],
        out_specs=[
            pl.BlockSpec((RA, HD), lambda i: (i, 0)),
            pl.BlockSpec((RA, HD), lambda i: (i, 0)),
        ],
        out_shape=[
            jax.ShapeDtypeStruct((N, HD), jnp.float32),
            jax.ShapeDtypeStruct((N, HD), jnp.float32),
        ],
    )(x, wa, wb, b2)


def _sc_tile_run(tab_hbm, agg_hbm, src_hbm, dst_hbm, srcb, dstb, rows,
                 stage, acc, sem_i, sem_r, s):
    z = jnp.zeros((16,), jnp.float32)

    @pl.loop(0, ZROWS)
    def _zero(r):
        stage[r, pl.ds(0, 16)] = z
        stage[r, pl.ds(16, 16)] = z

    zbase = s * TROWS
    for k in range(TROWS // ZROWS):
        pltpu.sync_copy(stage, acc.at[pl.ds(zbase + k * ZROWS, ZROWS)])
    plsc.subcore_barrier()

    ebase = s * (KCH * CHUNK)

    @pl.loop(0, KCH, step=NBUF)
    def _group(k0):
        idx_d = []
        for b in range(NBUF):
            off = ebase + (k0 + b) * CHUNK
            idx_d.append(pltpu.async_copy(
                src_hbm.at[pl.ds(off, CHUNK)], srcb[b], sem_i))
            idx_d.append(pltpu.async_copy(
                dst_hbm.at[pl.ds(off, CHUNK)], dstb[b], sem_i))
        for d in idx_d:
            d.wait()
        row_d = []
        for b in range(NBUF):
            row_d.append(pltpu.async_copy(
                tab_hbm.at[srcb[b]], rows[b], sem_r))
        for d in row_d:
            d.wait()
        for b in range(NBUF):
            pltpu.sync_copy(rows[b], acc.at[dstb[b]], add=True)

    plsc.subcore_barrier()

    obase = s * TROWS
    pltpu.sync_copy(acc.at[pl.ds(obase, TROWS)],
                    agg_hbm.at[pl.ds(obase, TROWS)])


def _sc_body(lo_hbm, hi_hbm, src_hbm, dst_hbm, agg_lo_hbm, agg_hi_hbm,
             *scratch):
    srcb = list(scratch[0:NBUF])
    dstb = list(scratch[NBUF:2 * NBUF])
    rows = list(scratch[2 * NBUF:3 * NBUF])
    stage, acc, sem_i, sem_r = scratch[3 * NBUF:]
    c = lax.axis_index("c")
    s = lax.axis_index("s")

    @pl.when(c == 0)
    def _():
        _sc_tile_run(lo_hbm, agg_lo_hbm, src_hbm, dst_hbm, srcb, dstb,
                     rows, stage, acc, sem_i, sem_r, s)

    @pl.when(c == 1)
    def _():
        _sc_tile_run(hi_hbm, agg_hi_hbm, src_hbm, dst_hbm, srcb, dstb,
                     rows, stage, acc, sem_i, sem_r, s)


def _edge_agg_sc(out_lo, out_hi, src_p, dst_p):
    mesh = plsc.VectorSubcoreMesh(core_axis_name="c", subcore_axis_name="s",
                                  num_cores=NC, num_subcores=NS)
    f = pl.kernel(
        _sc_body,
        out_type=[
            jax.ShapeDtypeStruct((ACC_ROWS, HD), jnp.float32),
            jax.ShapeDtypeStruct((ACC_ROWS, HD), jnp.float32),
        ],
        mesh=mesh,
        compiler_params=pltpu.CompilerParams(use_tc_tiling_on_sc=False),
        scratch_types=(
            [pltpu.VMEM((CHUNK,), jnp.int32) for _ in range(2 * NBUF)]
            + [pltpu.VMEM((CHUNK, HD), jnp.float32) for _ in range(NBUF)]
            + [
                pltpu.VMEM((ZROWS, HD), jnp.float32),
                pltpu.VMEM_SHARED((ACC_ROWS, HD), jnp.float32),
                pltpu.SemaphoreType.DMA,
                pltpu.SemaphoreType.DMA,
            ]
        ),
    )
    return f(out_lo, out_hi, src_p, dst_p)


def _conv_body(lo_ref, hi_ref, alo_ref, ahi_ref, w1_ref, w2_ref, b_ref,
               out_ref):
    hlo = lo_ref[...] + alo_ref[...]
    hhi = hi_ref[...] + ahi_ref[...]
    y = lax.dot_general(hlo, w1_ref[...], (((1,), (1,)), ((), ())),
                        preferred_element_type=jnp.float32,
                         precision=lax.Precision.DEFAULT)
    y = y + lax.dot_general(hhi, w2_ref[...], (((1,), (1,)), ((), ())),
                            preferred_element_type=jnp.float32,
                         precision=lax.Precision.DEFAULT)
    out_ref[...] = jnp.maximum(y + b_ref[...], 0.0)


def _conv(out_lo, out_hi, agg_lo, agg_hi, conv_W, conv_b):
    w1 = conv_W[:, 0:HD]
    w2 = conv_W[:, HD:DIM]
    b2 = conv_b.reshape(1, DIM)
    return pl.pallas_call(
        _conv_body,
        grid=(NBA,),
        in_specs=[
            pl.BlockSpec((RA, HD), lambda i: (i, 0)),
            pl.BlockSpec((RA, HD), lambda i: (i, 0)),
            pl.BlockSpec((RA, HD), lambda i: (i, 0)),
            pl.BlockSpec((RA, HD), lambda i: (i, 0)),
            pl.BlockSpec((DIM, HD), lambda i: (0, 0)),
            pl.BlockSpec((DIM, HD), lambda i: (0, 0)),
            pl.BlockSpec((1, DIM), lambda i: (0, 0)),
        ],
        out_specs=pl.BlockSpec((RA, DIM), lambda i: (i, 0)),
        out_shape=jax.ShapeDtypeStruct((N, DIM), jnp.float32),
    )(out_lo, out_hi, agg_lo, agg_hi, w1, w2, b2)


def _set2set_body(x_ref, bt_ref, wih_ref, whh_ref, bg_ref, w1_ref, b1_ref,
                  w2_ref, b2_ref, y_ref,
                  m_ref, ss_ref, r_ref, h_ref, c_ref, qs_ref):
    g = pl.program_id(0)
    j = g % NBC
    is_head = j == 0

    @pl.when(g == 0)
    def _init():
        qs_ref[...] = jnp.zeros_like(qs_ref)
        h_ref[...] = jnp.zeros_like(h_ref)
        c_ref[...] = jnp.zeros_like(c_ref)

    @pl.when(jnp.logical_and(is_head, g > 0))
    def _finalize():
        rv = r_ref[...] / (ss_ref[...] + 1e-16)
        qs_ref[:, 0:DIM] = h_ref[...]
        qs_ref[:, DIM:2 * DIM] = rv

    @pl.when(jnp.logical_and(is_head, g < G_GRID - 1))
    def _lstm():
        gates = lax.dot_general(qs_ref[...], wih_ref[...],
                                (((1,), (1,)), ((), ())),
                                preferred_element_type=jnp.float32,
                         precision=lax.Precision.DEFAULT)
        gates = gates + lax.dot_general(h_ref[...], whh_ref[...],
                                        (((1,), (1,)), ((), ())),
                                        preferred_element_type=jnp.float32,
                         precision=lax.Precision.DEFAULT)
        gates = gates + bg_ref[...]
        ig = jax.nn.sigmoid(gates[:, 0:DIM])
        fg = jax.nn.sigmoid(gates[:, DIM:2 * DIM])
        gg = jnp.tanh(gates[:, 2 * DIM:3 * DIM])
        og = jax.nn.sigmoid(gates[:, 3 * DIM:4 * DIM])
        cc = fg * c_ref[...] + ig * gg
        c_ref[...] = cc
        h_ref[...] = og * jnp.tanh(cc)
        m_ref[...] = jnp.full_like(m_ref, -1e30)
        ss_ref[...] = jnp.zeros_like(ss_ref)
        r_ref[...] = jnp.zeros_like(r_ref)

    @pl.when(g < G_GRID - 1)
    def _block():
        x = x_ref[...]
        bt = bt_ref[0]
        q = h_ref[...]
        et = lax.dot_general(q, x, (((1,), (1,)), ((), ())),
                             preferred_element_type=jnp.float32,
                             precision=lax.Precision.HIGHEST)
        iot = lax.broadcasted_iota(jnp.int32, (B, RC), 0)
        oh = iot == bt
        em = jnp.where(oh, et, -1e30)
        m_old = m_ref[...]
        m_new = jnp.maximum(m_old, jnp.max(em, axis=1, keepdims=True))
        scale = jnp.exp(m_old - m_new)
        p = jnp.where(oh, jnp.exp(em - m_new), 0.0)
        ss_ref[...] = ss_ref[...] * scale + jnp.sum(p, axis=1, keepdims=True)
        r_ref[...] = r_ref[...] * scale + lax.dot_general(
            p, x, (((1,), (0,)), ((), ())),
            preferred_element_type=jnp.float32,
            precision=lax.Precision.HIGHEST)
        m_ref[...] = m_new

    @pl.when(g == G_GRID - 1)
    def _out():
        y1 = lax.dot_general(qs_ref[...], w1_ref[...],
                             (((1,), (1,)), ((), ())),
                             preferred_element_type=jnp.float32,
                         precision=lax.Precision.DEFAULT)
        y1 = jnp.maximum(y1 + b1_ref[...], 0.0)
        yt = lax.dot_general(w2_ref[...], y1, (((1,), (1,)), ((), ())),
                             preferred_element_type=jnp.float32,
                         precision=lax.Precision.DEFAULT)
        y_ref[...] = yt + b2_ref[...]


def _set2set(out2, bt3, W_ih, W_hh, b_ih, b_hh, lin1_W, lin1_b, lin2_W,
             lin2_b):
    bg = (b_ih + b_hh).reshape(1, 4 * DIM)
    b1 = lin1_b.reshape(1, DIM)
    b2 = lin2_b.reshape(1, 1)
    y = pl.pallas_call(
        _set2set_body,
        grid=(G_GRID,),
        in_specs=[
            pl.BlockSpec((RC, DIM), lambda g: (g % NBC, 0)),
            pl.BlockSpec((1, 1, RC), lambda g: (g % NBC, 0, 0)),
            pl.BlockSpec((4 * DIM, 2 * DIM), lambda g: (0, 0)),
            pl.BlockSpec((4 * DIM, DIM), lambda g: (0, 0)),
            pl.BlockSpec((1, 4 * DIM), lambda g: (0, 0)),
            pl.BlockSpec((DIM, 2 * DIM), lambda g: (0, 0)),
            pl.BlockSpec((1, DIM), lambda g: (0, 0)),
            pl.BlockSpec((1, DIM), lambda g: (0, 0)),
            pl.BlockSpec((1, 1), lambda g: (0, 0)),
        ],
        out_specs=pl.BlockSpec((1, B), lambda g: (0, 0)),
        out_shape=jax.ShapeDtypeStruct((1, B), jnp.float32),
        scratch_shapes=[
            pltpu.VMEM((B, 1), jnp.float32),
            pltpu.VMEM((B, 1), jnp.float32),
            pltpu.VMEM((B, DIM), jnp.float32),
            pltpu.VMEM((B, DIM), jnp.float32),
            pltpu.VMEM((B, DIM), jnp.float32),
            pltpu.VMEM((B, 2 * DIM), jnp.float32),
        ],
    )(out2, bt3, W_ih, W_hh, bg, lin1_W, b1, lin2_W, b2)
    return y.reshape(-1)


def kernel(x, edge_index, batch, lin0_W, lin0_b, conv_W, conv_b, W_ih, W_hh,
           b_ih, b_hh, lin1_W, lin1_b, lin2_W, lin2_b):
    src = edge_index[0].astype(jnp.int32)
    dst = edge_index[1].astype(jnp.int32)
    pad = E_PAD - E
    apad = jnp.arange(pad, dtype=jnp.int32)
    src_p = jnp.concatenate([src, (apad * 97) % N])
    dst_p = jnp.concatenate([dst, N + (apad % NS)])
    bt3 = batch.astype(jnp.int32).reshape(NBC, 1, RC)

    out_lo, out_hi = _lin0(x, lin0_W, lin0_b)
    agg_lo, agg_hi = _edge_agg_sc(out_lo, out_hi, src_p, dst_p)
    out2 = _conv(out_lo, out_hi, agg_lo, agg_hi, conv_W, conv_b)
    return _set2set(out2, bt3, W_ih, W_hh, b_ih, b_hh, lin1_W, lin1_b,
                    lin2_W, lin2_b)


# speedup vs baseline: 11.7646x; 1.1872x over previous
import functools

import jax
import jax.numpy as jnp
from jax import lax
from jax.experimental import pallas as pl
from jax.experimental.pallas import tpu as pltpu
from jax.experimental.pallas import tpu_sc as plsc

N = 50000
E = 800000
MOL_IN = 25
DIM = 64
HD = 32
B = 128
STEPS = 3

NC = 2
NS = 16
CHUNK = 128
GROUP = 2
NBANK = 3
NG = 196
KCH = NG * GROUP
E_PAD = NS * CHUNK * KCH
ACC_ROWS = 50048
TROWS = ACC_ROWS // NS

RC = 2000
NBC = N // RC
G_GRID = STEPS * NBC + 1

RA = 2000
NBA = N // RA


def _lin0_body(x_ref, wa_ref, wb_ref, b_ref, lo_ref, hi_ref):
    x = x_ref[...]
    b = b_ref[...]
    lo = lax.dot_general(x, wa_ref[...], (((1,), (1,)), ((), ())),
                         preferred_element_type=jnp.float32,
                         precision=lax.Precision.DEFAULT)
    hi = lax.dot_general(x, wb_ref[...], (((1,), (1,)), ((), ())),
                         preferred_element_type=jnp.float32,
                         precision=lax.Precision.DEFAULT)
    lo_ref[...] = jnp.maximum(lo + b[:, 0:HD], 0.0)
    hi_ref[...] = jnp.maximum(hi + b[:, HD:DIM], 0.0)


def _lin0(x, lin0_W, lin0_b):
    wa = lin0_W[0:HD, :]
    wb = lin0_W[HD:DIM, :]
    b2 = lin0_b.reshape(1, DIM)
    return pl.pallas_call(
        _lin0_body,
        grid=(NBA,),
        in_specs=[
            pl.BlockSpec((RA, MOL_IN), lambda i: (i, 0)),
            pl.BlockSpec((HD, MOL_IN), lambda i: (0, 0)),
            pl.BlockSpec((HD, MOL_IN), lambda i: (0, 0)),
            pl.BlockSpec((1, DIM), lambda i: (0, 0)),
        ],
        out_specs=[
            pl.BlockSpec((RA, HD), lambda i: (i, 0)),
            pl.BlockSpec((RA, HD), lambda i: (i, 0)),
        ],
        out_shape=[
            jax.ShapeDtypeStruct((N, HD), jnp.float32),
            jax.ShapeDtypeStruct((N, HD), jnp.float32),
        ],
    )(x, wa, wb, b2)


def _sc_tile_run(tab_hbm, agg_hbm, src2_hbm, dst2_hbm, srcb, dstb, rows,
                 acc, sem_i, sem_r, sem_w, s):
    obase = s * TROWS

    @pl.when(s < NS - 1)
    def _():
        pltpu.sync_copy(tab_hbm.at[pl.ds(obase, TROWS)],
                        acc.at[pl.ds(obase, TROWS)])

    @pl.when(s == NS - 1)
    def _():
        pltpu.sync_copy(tab_hbm.at[pl.ds(obase, N - (NS - 1) * TROWS)],
                        acc.at[pl.ds(obase, N - (NS - 1) * TROWS)])

    plsc.subcore_barrier()

    cbase = s * KCH

    def issue_idx(g, k):
        off = (cbase + g * GROUP) * CHUNK
        for b in range(GROUP):
            pltpu.async_copy(src2_hbm.at[pl.ds(off + b * CHUNK, CHUNK)],
                             srcb[k][b], sem_i)
            pltpu.async_copy(dst2_hbm.at[pl.ds(off + b * CHUNK, CHUNK)],
                             dstb[k][b], sem_i)

    def wait_idx(k):
        for b in range(GROUP):
            pltpu.make_async_copy(src2_hbm.at[pl.ds(0, CHUNK)],
                                  srcb[k][b], sem_i).wait()
            pltpu.make_async_copy(dst2_hbm.at[pl.ds(0, CHUNK)],
                                  dstb[k][b], sem_i).wait()

    def issue_gather(k):
        for b in range(GROUP):
            pltpu.async_copy(tab_hbm.at[srcb[k][b]], rows[k][b], sem_r)

    def wait_gather(k):
        for b in range(GROUP):
            pltpu.make_async_copy(tab_hbm.at[srcb[k][b]], rows[k][b],
                                  sem_r).wait()

    def issue_scatter(k):
        for b in range(GROUP):
            pltpu.async_copy(rows[k][b], acc.at[dstb[k][b]], sem_w,
                             add=True)

    def wait_scatter(k):
        for b in range(GROUP):
            pltpu.make_async_copy(rows[k][b], acc.at[dstb[k][b]],
                                  sem_w).wait()

    issue_idx(0, 0)
    wait_idx(0); issue_gather(0); issue_idx(1, 1)
    wait_idx(1); issue_gather(1); issue_idx(2, 2)
    wait_gather(0); issue_scatter(0)
    wait_idx(2); issue_gather(2)
    wait_scatter(0); issue_idx(3, 0)
    wait_gather(1); issue_scatter(1)

    @pl.loop(0, (NG - 4) // NBANK)
    def _main(go):
        for u in range(NBANK):
            g = 3 + go * NBANK + u
            k, kprev, knext = u, (u - 1) % NBANK, (u + 1) % NBANK
            wait_idx(k); issue_gather(k)
            wait_gather(kprev); issue_scatter(kprev)
            wait_scatter(knext); issue_idx(g + 1, knext)

    wait_idx(0); issue_gather(0)
    wait_gather(2); issue_scatter(2)
    wait_gather(0); issue_scatter(0)
    wait_scatter(1); wait_scatter(2); wait_scatter(0)

    plsc.subcore_barrier()

    pltpu.sync_copy(acc.at[pl.ds(obase, TROWS)],
                    agg_hbm.at[pl.ds(obase, TROWS)])


def _sc_body(lo_hbm, hi_hbm, src2_hbm, dst2_hbm, agg_lo_hbm, agg_hi_hbm,
             *scratch):
    g3 = NBANK * GROUP
    srcb = [list(scratch[k * GROUP:(k + 1) * GROUP]) for k in range(NBANK)]
    dstb = [list(scratch[g3 + k * GROUP:g3 + (k + 1) * GROUP])
            for k in range(NBANK)]
    rows = [list(scratch[2 * g3 + k * GROUP:2 * g3 + (k + 1) * GROUP])
            for k in range(NBANK)]
    acc, sem_i, sem_r, sem_w = scratch[3 * g3:]
    c = lax.axis_index("c")
    s = lax.axis_index("s")

    @pl.when(c == 0)
    def _():
        _sc_tile_run(lo_hbm, agg_lo_hbm, src2_hbm, dst2_hbm, srcb, dstb,
                     rows, acc, sem_i, sem_r, sem_w, s)

    @pl.when(c == 1)
    def _():
        _sc_tile_run(hi_hbm, agg_hi_hbm, src2_hbm, dst2_hbm, srcb, dstb,
                     rows, acc, sem_i, sem_r, sem_w, s)


def _edge_agg_sc(out_lo, out_hi, src2, dst2):
    mesh = plsc.VectorSubcoreMesh(core_axis_name="c", subcore_axis_name="s",
                                  num_cores=NC, num_subcores=NS)
    f = pl.kernel(
        _sc_body,
        out_type=[
            jax.ShapeDtypeStruct((ACC_ROWS, HD), jnp.float32),
            jax.ShapeDtypeStruct((ACC_ROWS, HD), jnp.float32),
        ],
        mesh=mesh,
        compiler_params=pltpu.CompilerParams(use_tc_tiling_on_sc=False),
        scratch_types=(
            [pltpu.VMEM((CHUNK,), jnp.int32)
             for _ in range(2 * NBANK * GROUP)]
            + [pltpu.VMEM((CHUNK, HD), jnp.float32)
               for _ in range(NBANK * GROUP)]
            + [
                pltpu.VMEM_SHARED((ACC_ROWS, HD), jnp.float32),
                pltpu.SemaphoreType.DMA,
                pltpu.SemaphoreType.DMA,
                pltpu.SemaphoreType.DMA,
            ]
        ),
    )
    return f(out_lo, out_hi, src2, dst2)


def _conv_body(hlo_ref, hhi_ref, w1_ref, w2_ref, b_ref, out_ref):
    hlo = hlo_ref[...]
    hhi = hhi_ref[...]
    y = lax.dot_general(hlo, w1_ref[...], (((1,), (1,)), ((), ())),
                        preferred_element_type=jnp.float32,
                         precision=lax.Precision.DEFAULT)
    y = y + lax.dot_general(hhi, w2_ref[...], (((1,), (1,)), ((), ())),
                            preferred_element_type=jnp.float32,
                         precision=lax.Precision.DEFAULT)
    out_ref[...] = jnp.maximum(y + b_ref[...], 0.0)


def _conv(h_lo, h_hi, conv_W, conv_b):
    w1 = conv_W[:, 0:HD]
    w2 = conv_W[:, HD:DIM]
    b2 = conv_b.reshape(1, DIM)
    return pl.pallas_call(
        _conv_body,
        grid=(NBA,),
        in_specs=[
            pl.BlockSpec((RA, HD), lambda i: (i, 0)),
            pl.BlockSpec((RA, HD), lambda i: (i, 0)),
            pl.BlockSpec((DIM, HD), lambda i: (0, 0)),
            pl.BlockSpec((DIM, HD), lambda i: (0, 0)),
            pl.BlockSpec((1, DIM), lambda i: (0, 0)),
        ],
        out_specs=pl.BlockSpec((RA, DIM), lambda i: (i, 0)),
        out_shape=jax.ShapeDtypeStruct((N, DIM), jnp.float32),
    )(h_lo, h_hi, w1, w2, b2)


def _set2set_body(x_ref, bt_ref, wih_ref, whh_ref, bg_ref, w1_ref, b1_ref,
                  w2_ref, b2_ref, y_ref,
                  m_ref, ss_ref, r_ref, h_ref, c_ref, qs_ref):
    g = pl.program_id(0)
    j = g % NBC
    is_head = j == 0

    @pl.when(g == 0)
    def _init():
        qs_ref[...] = jnp.zeros_like(qs_ref)
        h_ref[...] = jnp.zeros_like(h_ref)
        c_ref[...] = jnp.zeros_like(c_ref)

    @pl.when(jnp.logical_and(is_head, g > 0))
    def _finalize():
        rv = r_ref[...] / (ss_ref[...] + 1e-16)
        qs_ref[:, 0:DIM] = h_ref[...]
        qs_ref[:, DIM:2 * DIM] = rv

    @pl.when(jnp.logical_and(is_head, g < G_GRID - 1))
    def _lstm():
        gates = lax.dot_general(qs_ref[...], wih_ref[...],
                                (((1,), (1,)), ((), ())),
                                preferred_element_type=jnp.float32,
                         precision=lax.Precision.DEFAULT)
        gates = gates + lax.dot_general(h_ref[...], whh_ref[...],
                                        (((1,), (1,)), ((), ())),
                                        preferred_element_type=jnp.float32,
                         precision=lax.Precision.DEFAULT)
        gates = gates + bg_ref[...]
        ig = jax.nn.sigmoid(gates[:, 0:DIM])
        fg = jax.nn.sigmoid(gates[:, DIM:2 * DIM])
        gg = jnp.tanh(gates[:, 2 * DIM:3 * DIM])
        og = jax.nn.sigmoid(gates[:, 3 * DIM:4 * DIM])
        cc = fg * c_ref[...] + ig * gg
        c_ref[...] = cc
        h_ref[...] = og * jnp.tanh(cc)
        m_ref[...] = jnp.full_like(m_ref, -1e30)
        ss_ref[...] = jnp.zeros_like(ss_ref)
        r_ref[...] = jnp.zeros_like(r_ref)

    @pl.when(g < G_GRID - 1)
    def _block():
        x = x_ref[...]
        bt = bt_ref[0]
        q = h_ref[...]
        et = lax.dot_general(q, x, (((1,), (1,)), ((), ())),
                             preferred_element_type=jnp.float32,
                             precision=lax.Precision.HIGHEST)
        iot = lax.broadcasted_iota(jnp.int32, (B, RC), 0)
        oh = iot == bt
        em = jnp.where(oh, et, -1e30)
        m_old = m_ref[...]
        m_new = jnp.maximum(m_old, jnp.max(em, axis=1, keepdims=True))
        scale = jnp.exp(m_old - m_new)
        p = jnp.where(oh, jnp.exp(em - m_new), 0.0)
        ss_ref[...] = ss_ref[...] * scale + jnp.sum(p, axis=1, keepdims=True)
        r_ref[...] = r_ref[...] * scale + lax.dot_general(
            p, x, (((1,), (0,)), ((), ())),
            preferred_element_type=jnp.float32,
            precision=lax.Precision.HIGHEST)
        m_ref[...] = m_new

    @pl.when(g == G_GRID - 1)
    def _out():
        y1 = lax.dot_general(qs_ref[...], w1_ref[...],
                             (((1,), (1,)), ((), ())),
                             preferred_element_type=jnp.float32,
                         precision=lax.Precision.DEFAULT)
        y1 = jnp.maximum(y1 + b1_ref[...], 0.0)
        yt = lax.dot_general(w2_ref[...], y1, (((1,), (1,)), ((), ())),
                             preferred_element_type=jnp.float32,
                         precision=lax.Precision.DEFAULT)
        y_ref[...] = yt + b2_ref[...]


def _set2set(out2, bt3, W_ih, W_hh, b_ih, b_hh, lin1_W, lin1_b, lin2_W,
             lin2_b):
    bg = (b_ih + b_hh).reshape(1, 4 * DIM)
    b1 = lin1_b.reshape(1, DIM)
    b2 = lin2_b.reshape(1, 1)
    y = pl.pallas_call(
        _set2set_body,
        grid=(G_GRID,),
        in_specs=[
            pl.BlockSpec((RC, DIM), lambda g: (g % NBC, 0)),
            pl.BlockSpec((1, 1, RC), lambda g: (g % NBC, 0, 0)),
            pl.BlockSpec((4 * DIM, 2 * DIM), lambda g: (0, 0)),
            pl.BlockSpec((4 * DIM, DIM), lambda g: (0, 0)),
            pl.BlockSpec((1, 4 * DIM), lambda g: (0, 0)),
            pl.BlockSpec((DIM, 2 * DIM), lambda g: (0, 0)),
            pl.BlockSpec((1, DIM), lambda g: (0, 0)),
            pl.BlockSpec((1, DIM), lambda g: (0, 0)),
            pl.BlockSpec((1, 1), lambda g: (0, 0)),
        ],
        out_specs=pl.BlockSpec((1, B), lambda g: (0, 0)),
        out_shape=jax.ShapeDtypeStruct((1, B), jnp.float32),
        scratch_shapes=[
            pltpu.VMEM((B, 1), jnp.float32),
            pltpu.VMEM((B, 1), jnp.float32),
            pltpu.VMEM((B, DIM), jnp.float32),
            pltpu.VMEM((B, DIM), jnp.float32),
            pltpu.VMEM((B, DIM), jnp.float32),
            pltpu.VMEM((B, 2 * DIM), jnp.float32),
        ],
    )(out2, bt3, W_ih, W_hh, bg, lin1_W, b1, lin2_W, b2)
    return y.reshape(-1)


def kernel(x, edge_index, batch, lin0_W, lin0_b, conv_W, conv_b, W_ih, W_hh,
           b_ih, b_hh, lin1_W, lin1_b, lin2_W, lin2_b):
    src = edge_index[0].astype(jnp.int32)
    dst = edge_index[1].astype(jnp.int32)
    pad = E_PAD - E
    apad = jnp.arange(pad, dtype=jnp.int32)
    src2 = jnp.concatenate([src, (apad * 97) % N])
    dst2 = jnp.concatenate([dst, N + (apad % NS)])
    bt3 = batch.astype(jnp.int32).reshape(NBC, 1, RC)

    out_lo, out_hi = _lin0(x, lin0_W, lin0_b)
    h_lo, h_hi = _edge_agg_sc(out_lo, out_hi, src2, dst2)
    out2 = _conv(h_lo, h_hi, conv_W, conv_b)
    return _set2set(out2, bt3, W_ih, W_hh, b_ih, b_hh, lin1_W, lin1_b,
                    lin2_W, lin2_b)


# speedup vs baseline: 11.8550x; 1.0077x over previous
import functools

import jax
import jax.numpy as jnp
from jax import lax
from jax.experimental import pallas as pl
from jax.experimental.pallas import tpu as pltpu
from jax.experimental.pallas import tpu_sc as plsc

N = 50000
E = 800000
MOL_IN = 25
DIM = 64
HD = 32
B = 128
STEPS = 3

NC = 2
NS = 16
CHUNK = 128
GROUP = 2
NBANK = 3
NG = 196
KCH = NG * GROUP
E_PAD = NS * CHUNK * KCH
ACC_ROWS = 50048
TROWS = ACC_ROWS // NS

RC = 2000
NBC = N // RC
G_GRID = STEPS * NBC + 1

RA = 2000
NBA = N // RA


def _lin0_body(x_ref, wa_ref, wb_ref, b_ref, lo_ref, hi_ref):
    x = x_ref[...]
    b = b_ref[...]
    lo = lax.dot_general(x, wa_ref[...], (((1,), (1,)), ((), ())),
                         preferred_element_type=jnp.float32,
                         precision=lax.Precision.DEFAULT)
    hi = lax.dot_general(x, wb_ref[...], (((1,), (1,)), ((), ())),
                         preferred_element_type=jnp.float32,
                         precision=lax.Precision.DEFAULT)
    lo_ref[...] = jnp.maximum(lo + b[:, 0:HD], 0.0)
    hi_ref[...] = jnp.maximum(hi + b[:, HD:DIM], 0.0)


def _lin0(x, lin0_W, lin0_b):
    wa = lin0_W[0:HD, :]
    wb = lin0_W[HD:DIM, :]
    b2 = lin0_b.reshape(1, DIM)
    return pl.pallas_call(
        _lin0_body,
        grid=(NBA,),
        in_specs=[
            pl.BlockSpec((RA, MOL_IN), lambda i: (i, 0)),
            pl.BlockSpec((HD, MOL_IN), lambda i: (0, 0)),
            pl.BlockSpec((HD, MOL_IN), lambda i: (0, 0)),
            pl.BlockSpec((1, DIM), lambda i: (0, 0)),
        ],
        out_specs=[
            pl.BlockSpec((RA, HD), lambda i: (i, 0)),
            pl.BlockSpec((RA, HD), lambda i: (i, 0)),
        ],
        out_shape=[
            jax.ShapeDtypeStruct((N, HD), jnp.float32),
            jax.ShapeDtypeStruct((N, HD), jnp.float32),
        ],
    )(x, wa, wb, b2)


def _sc_tile_run(tab_hbm, agg_hbm, src2_hbm, dst2_hbm, srcb, dstb, rows,
                 acc, sem_i, sem_r, sem_w, s):
    obase = s * TROWS

    @pl.when(s < NS - 1)
    def _():
        pltpu.sync_copy(tab_hbm.at[pl.ds(obase, TROWS)],
                        acc.at[pl.ds(obase, TROWS)])

    @pl.when(s == NS - 1)
    def _():
        pltpu.sync_copy(tab_hbm.at[pl.ds(obase, N - (NS - 1) * TROWS)],
                        acc.at[pl.ds(obase, N - (NS - 1) * TROWS)])

    plsc.subcore_barrier()

    cbase = s * KCH

    def issue_idx(g, k):
        off = (cbase + g * GROUP) * CHUNK
        for b in range(GROUP):
            pltpu.async_copy(src2_hbm.at[pl.ds(off + b * CHUNK, CHUNK)],
                             srcb[k][b], sem_i)
            pltpu.async_copy(dst2_hbm.at[pl.ds(off + b * CHUNK, CHUNK)],
                             dstb[k][b], sem_i)

    def wait_idx(k):
        for b in range(GROUP):
            pltpu.make_async_copy(src2_hbm.at[pl.ds(0, CHUNK)],
                                  srcb[k][b], sem_i).wait()
            pltpu.make_async_copy(dst2_hbm.at[pl.ds(0, CHUNK)],
                                  dstb[k][b], sem_i).wait()

    def issue_gather(k):
        for b in range(GROUP):
            pltpu.async_copy(tab_hbm.at[srcb[k][b]], rows[k][b], sem_r)

    def wait_gather(k):
        for b in range(GROUP):
            pltpu.make_async_copy(tab_hbm.at[srcb[k][b]], rows[k][b],
                                  sem_r).wait()

    def issue_scatter(k):
        for b in range(GROUP):
            pltpu.async_copy(rows[k][b], acc.at[dstb[k][b]], sem_w,
                             add=True)

    def wait_scatter(k):
        for b in range(GROUP):
            pltpu.make_async_copy(rows[k][b], acc.at[dstb[k][b]],
                                  sem_w).wait()

    issue_idx(0, 0)
    wait_idx(0); issue_gather(0); issue_idx(1, 1)
    wait_idx(1); issue_gather(1); issue_idx(2, 2)
    wait_gather(0); issue_scatter(0)
    wait_idx(2); issue_gather(2)
    wait_scatter(0); issue_idx(3, 0)
    wait_gather(1); issue_scatter(1)

    @pl.loop(0, (NG - 4) // NBANK)
    def _main(go):
        for u in range(NBANK):
            g = 3 + go * NBANK + u
            k, kprev, knext = u, (u - 1) % NBANK, (u + 1) % NBANK
            wait_idx(k); issue_gather(k)
            wait_gather(kprev); issue_scatter(kprev)
            wait_scatter(knext); issue_idx(g + 1, knext)

    wait_idx(0); issue_gather(0)
    wait_gather(2); issue_scatter(2)
    wait_gather(0); issue_scatter(0)
    wait_scatter(1); wait_scatter(2); wait_scatter(0)

    plsc.subcore_barrier()

    pltpu.sync_copy(acc.at[pl.ds(obase, TROWS)],
                    agg_hbm.at[pl.ds(obase, TROWS)])


def _sc_body(lo_hbm, hi_hbm, src2_hbm, dst2_hbm, agg_lo_hbm, agg_hi_hbm,
             *scratch):
    g3 = NBANK * GROUP
    srcb = [list(scratch[k * GROUP:(k + 1) * GROUP]) for k in range(NBANK)]
    dstb = [list(scratch[g3 + k * GROUP:g3 + (k + 1) * GROUP])
            for k in range(NBANK)]
    rows = [list(scratch[2 * g3 + k * GROUP:2 * g3 + (k + 1) * GROUP])
            for k in range(NBANK)]
    acc, sem_i, sem_r, sem_w = scratch[3 * g3:]
    c = lax.axis_index("c")
    s = lax.axis_index("s")

    @pl.when(c == 0)
    def _():
        _sc_tile_run(lo_hbm, agg_lo_hbm, src2_hbm, dst2_hbm, srcb, dstb,
                     rows, acc, sem_i, sem_r, sem_w, s)

    @pl.when(c == 1)
    def _():
        _sc_tile_run(hi_hbm, agg_hi_hbm, src2_hbm, dst2_hbm, srcb, dstb,
                     rows, acc, sem_i, sem_r, sem_w, s)


def _edge_agg_sc(out_lo, out_hi, src2, dst2):
    mesh = plsc.VectorSubcoreMesh(core_axis_name="c", subcore_axis_name="s",
                                  num_cores=NC, num_subcores=NS)
    f = pl.kernel(
        _sc_body,
        out_type=[
            jax.ShapeDtypeStruct((ACC_ROWS, HD), jnp.float32),
            jax.ShapeDtypeStruct((ACC_ROWS, HD), jnp.float32),
        ],
        mesh=mesh,
        compiler_params=pltpu.CompilerParams(use_tc_tiling_on_sc=False),
        scratch_types=(
            [pltpu.VMEM((CHUNK,), jnp.int32)
             for _ in range(2 * NBANK * GROUP)]
            + [pltpu.VMEM((CHUNK, HD), jnp.float32)
               for _ in range(NBANK * GROUP)]
            + [
                pltpu.VMEM_SHARED((ACC_ROWS, HD), jnp.float32),
                pltpu.SemaphoreType.DMA,
                pltpu.SemaphoreType.DMA,
                pltpu.SemaphoreType.DMA,
            ]
        ),
    )
    return f(out_lo, out_hi, src2, dst2)


def _conv_body(hlo_ref, hhi_ref, w1_ref, w2_ref, b_ref, out_ref):
    hlo = hlo_ref[...]
    hhi = hhi_ref[...]
    y = lax.dot_general(hlo, w1_ref[...], (((1,), (1,)), ((), ())),
                        preferred_element_type=jnp.float32,
                         precision=lax.Precision.DEFAULT)
    y = y + lax.dot_general(hhi, w2_ref[...], (((1,), (1,)), ((), ())),
                            preferred_element_type=jnp.float32,
                         precision=lax.Precision.DEFAULT)
    out_ref[...] = jnp.maximum(y + b_ref[...], 0.0)


def _conv(h_lo, h_hi, conv_W, conv_b):
    w1 = conv_W[:, 0:HD]
    w2 = conv_W[:, HD:DIM]
    b2 = conv_b.reshape(1, DIM)
    return pl.pallas_call(
        _conv_body,
        grid=(NBA,),
        in_specs=[
            pl.BlockSpec((RA, HD), lambda i: (i, 0)),
            pl.BlockSpec((RA, HD), lambda i: (i, 0)),
            pl.BlockSpec((DIM, HD), lambda i: (0, 0)),
            pl.BlockSpec((DIM, HD), lambda i: (0, 0)),
            pl.BlockSpec((1, DIM), lambda i: (0, 0)),
        ],
        out_specs=pl.BlockSpec((RA, DIM), lambda i: (i, 0)),
        out_shape=jax.ShapeDtypeStruct((N, DIM), jnp.float32),
    )(h_lo, h_hi, w1, w2, b2)


def _set2set_body(x_ref, bt_ref, wih_ref, whh_ref, bg_ref, w1_ref, b1_ref,
                  w2_ref, b2_ref, y_ref,
                  m_ref, ss_ref, r_ref, h_ref, c_ref, qs_ref):
    g = pl.program_id(0)
    j = g % NBC
    is_head = j == 0

    @pl.when(g == 0)
    def _init():
        qs_ref[...] = jnp.zeros_like(qs_ref)
        h_ref[...] = jnp.zeros_like(h_ref)
        c_ref[...] = jnp.zeros_like(c_ref)

    @pl.when(jnp.logical_and(is_head, g > 0))
    def _finalize():
        rv = r_ref[...] / (ss_ref[...] + 1e-16)
        qs_ref[:, 0:DIM] = h_ref[...]
        qs_ref[:, DIM:2 * DIM] = rv

    @pl.when(jnp.logical_and(is_head, g < G_GRID - 1))
    def _lstm():
        gates = lax.dot_general(qs_ref[...], wih_ref[...],
                                (((1,), (1,)), ((), ())),
                                preferred_element_type=jnp.float32,
                         precision=lax.Precision.DEFAULT)
        gates = gates + lax.dot_general(h_ref[...], whh_ref[...],
                                        (((1,), (1,)), ((), ())),
                                        preferred_element_type=jnp.float32,
                         precision=lax.Precision.DEFAULT)
        gates = gates + bg_ref[...]
        ig = jax.nn.sigmoid(gates[:, 0:DIM])
        fg = jax.nn.sigmoid(gates[:, DIM:2 * DIM])
        gg = jnp.tanh(gates[:, 2 * DIM:3 * DIM])
        og = jax.nn.sigmoid(gates[:, 3 * DIM:4 * DIM])
        cc = fg * c_ref[...] + ig * gg
        c_ref[...] = cc
        h_ref[...] = og * jnp.tanh(cc)
        m_ref[...] = jnp.full_like(m_ref, -1e30)
        ss_ref[...] = jnp.zeros_like(ss_ref)
        r_ref[...] = jnp.zeros_like(r_ref)

    @pl.when(g < G_GRID - 1)
    def _block():
        x = x_ref[...]
        bt = bt_ref[0]
        q = h_ref[...]
        et = lax.dot_general(q, x, (((1,), (1,)), ((), ())),
                             preferred_element_type=jnp.float32,
                             precision=lax.Precision.HIGHEST)
        iot = lax.broadcasted_iota(jnp.int32, (B, RC), 0)
        oh = iot == bt
        em = jnp.where(oh, et, -1e30)
        m_old = m_ref[...]
        m_new = jnp.maximum(m_old, jnp.max(em, axis=1, keepdims=True))
        scale = jnp.exp(m_old - m_new)
        p = jnp.exp(em - jnp.maximum(m_new, -1e20))
        ss_ref[...] = ss_ref[...] * scale + jnp.sum(p, axis=1, keepdims=True)
        r_ref[...] = r_ref[...] * scale + lax.dot_general(
            p, x, (((1,), (0,)), ((), ())),
            preferred_element_type=jnp.float32,
            precision=lax.Precision.HIGHEST)
        m_ref[...] = m_new

    @pl.when(g == G_GRID - 1)
    def _out():
        y1 = lax.dot_general(qs_ref[...], w1_ref[...],
                             (((1,), (1,)), ((), ())),
                             preferred_element_type=jnp.float32,
                         precision=lax.Precision.DEFAULT)
        y1 = jnp.maximum(y1 + b1_ref[...], 0.0)
        yt = lax.dot_general(w2_ref[...], y1, (((1,), (1,)), ((), ())),
                             preferred_element_type=jnp.float32,
                         precision=lax.Precision.DEFAULT)
        y_ref[...] = yt + b2_ref[...]


def _set2set(out2, bt3, W_ih, W_hh, b_ih, b_hh, lin1_W, lin1_b, lin2_W,
             lin2_b):
    bg = (b_ih + b_hh).reshape(1, 4 * DIM)
    b1 = lin1_b.reshape(1, DIM)
    b2 = lin2_b.reshape(1, 1)
    y = pl.pallas_call(
        _set2set_body,
        grid=(G_GRID,),
        in_specs=[
            pl.BlockSpec((RC, DIM), lambda g: (g % NBC, 0)),
            pl.BlockSpec((1, 1, RC), lambda g: (g % NBC, 0, 0)),
            pl.BlockSpec((4 * DIM, 2 * DIM), lambda g: (0, 0)),
            pl.BlockSpec((4 * DIM, DIM), lambda g: (0, 0)),
            pl.BlockSpec((1, 4 * DIM), lambda g: (0, 0)),
            pl.BlockSpec((DIM, 2 * DIM), lambda g: (0, 0)),
            pl.BlockSpec((1, DIM), lambda g: (0, 0)),
            pl.BlockSpec((1, DIM), lambda g: (0, 0)),
            pl.BlockSpec((1, 1), lambda g: (0, 0)),
        ],
        out_specs=pl.BlockSpec((1, B), lambda g: (0, 0)),
        out_shape=jax.ShapeDtypeStruct((1, B), jnp.float32),
        scratch_shapes=[
            pltpu.VMEM((B, 1), jnp.float32),
            pltpu.VMEM((B, 1), jnp.float32),
            pltpu.VMEM((B, DIM), jnp.float32),
            pltpu.VMEM((B, DIM), jnp.float32),
            pltpu.VMEM((B, DIM), jnp.float32),
            pltpu.VMEM((B, 2 * DIM), jnp.float32),
        ],
    )(out2, bt3, W_ih, W_hh, bg, lin1_W, b1, lin2_W, b2)
    return y.reshape(-1)


def kernel(x, edge_index, batch, lin0_W, lin0_b, conv_W, conv_b, W_ih, W_hh,
           b_ih, b_hh, lin1_W, lin1_b, lin2_W, lin2_b):
    src = edge_index[0].astype(jnp.int32)
    dst = edge_index[1].astype(jnp.int32)
    pad = E_PAD - E
    apad = jnp.arange(pad, dtype=jnp.int32)
    src2 = jnp.concatenate([src, (apad * 97) % N])
    dst2 = jnp.concatenate([dst, N + (apad % NS)])
    bt3 = batch.astype(jnp.int32).reshape(NBC, 1, RC)

    out_lo, out_hi = _lin0(x, lin0_W, lin0_b)
    h_lo, h_hi = _edge_agg_sc(out_lo, out_hi, src2, dst2)
    out2 = _conv(h_lo, h_hi, conv_W, conv_b)
    return _set2set(out2, bt3, W_ih, W_hh, b_ih, b_hh, lin1_W, lin1_b,
                    lin2_W, lin2_b)


# speedup vs baseline: 12.4086x; 1.0467x over previous
import functools

import jax
import jax.numpy as jnp
from jax import lax
from jax.experimental import pallas as pl
from jax.experimental.pallas import tpu as pltpu
from jax.experimental.pallas import tpu_sc as plsc

N = 50000
E = 800000
MOL_IN = 25
DIM = 64
HD = 32
B = 128
STEPS = 3

NC = 2
NS = 16
CHUNK = 128
GROUP = 2
NBANK = 3
NG = 196
KCH = NG * GROUP
E_PAD = NS * CHUNK * KCH
ACC_ROWS = 50048
TROWS = ACC_ROWS // NS

RC = 2000
NBC = N // RC
G_GRID = STEPS * NBC + 1

WSEG = 48

RA = 2000
NBA = N // RA


def _lin0_body(x_ref, wa_ref, wb_ref, b_ref, lo_ref, hi_ref):
    x = x_ref[...]
    b = b_ref[...]
    lo = lax.dot_general(x, wa_ref[...], (((1,), (1,)), ((), ())),
                         preferred_element_type=jnp.float32,
                         precision=lax.Precision.DEFAULT)
    hi = lax.dot_general(x, wb_ref[...], (((1,), (1,)), ((), ())),
                         preferred_element_type=jnp.float32,
                         precision=lax.Precision.DEFAULT)
    lo_ref[...] = jnp.maximum(lo + b[:, 0:HD], 0.0)
    hi_ref[...] = jnp.maximum(hi + b[:, HD:DIM], 0.0)


def _lin0(x, lin0_W, lin0_b):
    wa = lin0_W[0:HD, :]
    wb = lin0_W[HD:DIM, :]
    b2 = lin0_b.reshape(1, DIM)
    return pl.pallas_call(
        _lin0_body,
        grid=(NBA,),
        in_specs=[
            pl.BlockSpec((RA, MOL_IN), lambda i: (i, 0)),
            pl.BlockSpec((HD, MOL_IN), lambda i: (0, 0)),
            pl.BlockSpec((HD, MOL_IN), lambda i: (0, 0)),
            pl.BlockSpec((1, DIM), lambda i: (0, 0)),
        ],
        out_specs=[
            pl.BlockSpec((RA, HD), lambda i: (i, 0)),
            pl.BlockSpec((RA, HD), lambda i: (i, 0)),
        ],
        out_shape=[
            jax.ShapeDtypeStruct((N, HD), jnp.float32),
            jax.ShapeDtypeStruct((N, HD), jnp.float32),
        ],
    )(x, wa, wb, b2)


def _sc_tile_run(tab_hbm, agg_hbm, src2_hbm, dst2_hbm, srcb, dstb, rows,
                 acc, sem_i, sem_r, sem_w, s):
    obase = s * TROWS

    @pl.when(s < NS - 1)
    def _():
        pltpu.sync_copy(tab_hbm.at[pl.ds(obase, TROWS)],
                        acc.at[pl.ds(obase, TROWS)])

    @pl.when(s == NS - 1)
    def _():
        pltpu.sync_copy(tab_hbm.at[pl.ds(obase, N - (NS - 1) * TROWS)],
                        acc.at[pl.ds(obase, N - (NS - 1) * TROWS)])

    plsc.subcore_barrier()

    cbase = s * KCH

    def issue_idx(g, k):
        off = (cbase + g * GROUP) * CHUNK
        for b in range(GROUP):
            pltpu.async_copy(src2_hbm.at[pl.ds(off + b * CHUNK, CHUNK)],
                             srcb[k][b], sem_i)
            pltpu.async_copy(dst2_hbm.at[pl.ds(off + b * CHUNK, CHUNK)],
                             dstb[k][b], sem_i)

    def wait_idx(k):
        for b in range(GROUP):
            pltpu.make_async_copy(src2_hbm.at[pl.ds(0, CHUNK)],
                                  srcb[k][b], sem_i).wait()
            pltpu.make_async_copy(dst2_hbm.at[pl.ds(0, CHUNK)],
                                  dstb[k][b], sem_i).wait()

    def issue_gather(k):
        for b in range(GROUP):
            pltpu.async_copy(tab_hbm.at[srcb[k][b]], rows[k][b], sem_r)

    def wait_gather(k):
        for b in range(GROUP):
            pltpu.make_async_copy(tab_hbm.at[srcb[k][b]], rows[k][b],
                                  sem_r).wait()

    def issue_scatter(k):
        for b in range(GROUP):
            pltpu.async_copy(rows[k][b], acc.at[dstb[k][b]], sem_w,
                             add=True)

    def wait_scatter(k):
        for b in range(GROUP):
            pltpu.make_async_copy(rows[k][b], acc.at[dstb[k][b]],
                                  sem_w).wait()

    issue_idx(0, 0)
    wait_idx(0); issue_gather(0); issue_idx(1, 1)
    wait_idx(1); issue_gather(1); issue_idx(2, 2)
    wait_gather(0); issue_scatter(0)
    wait_idx(2); issue_gather(2)
    wait_scatter(0); issue_idx(3, 0)
    wait_gather(1); issue_scatter(1)

    @pl.loop(0, (NG - 4) // NBANK)
    def _main(go):
        for u in range(NBANK):
            g = 3 + go * NBANK + u
            k, kprev, knext = u, (u - 1) % NBANK, (u + 1) % NBANK
            wait_idx(k); issue_gather(k)
            wait_gather(kprev); issue_scatter(kprev)
            wait_scatter(knext); issue_idx(g + 1, knext)

    wait_idx(0); issue_gather(0)
    wait_gather(2); issue_scatter(2)
    wait_gather(0); issue_scatter(0)
    wait_scatter(1); wait_scatter(2); wait_scatter(0)

    plsc.subcore_barrier()

    pltpu.sync_copy(acc.at[pl.ds(obase, TROWS)],
                    agg_hbm.at[pl.ds(obase, TROWS)])


def _sc_body(lo_hbm, hi_hbm, src2_hbm, dst2_hbm, agg_lo_hbm, agg_hi_hbm,
             *scratch):
    g3 = NBANK * GROUP
    srcb = [list(scratch[k * GROUP:(k + 1) * GROUP]) for k in range(NBANK)]
    dstb = [list(scratch[g3 + k * GROUP:g3 + (k + 1) * GROUP])
            for k in range(NBANK)]
    rows = [list(scratch[2 * g3 + k * GROUP:2 * g3 + (k + 1) * GROUP])
            for k in range(NBANK)]
    acc, sem_i, sem_r, sem_w = scratch[3 * g3:]
    c = lax.axis_index("c")
    s = lax.axis_index("s")

    @pl.when(c == 0)
    def _():
        _sc_tile_run(lo_hbm, agg_lo_hbm, src2_hbm, dst2_hbm, srcb, dstb,
                     rows, acc, sem_i, sem_r, sem_w, s)

    @pl.when(c == 1)
    def _():
        _sc_tile_run(hi_hbm, agg_hi_hbm, src2_hbm, dst2_hbm, srcb, dstb,
                     rows, acc, sem_i, sem_r, sem_w, s)


def _edge_agg_sc(out_lo, out_hi, src2, dst2):
    mesh = plsc.VectorSubcoreMesh(core_axis_name="c", subcore_axis_name="s",
                                  num_cores=NC, num_subcores=NS)
    f = pl.kernel(
        _sc_body,
        out_type=[
            jax.ShapeDtypeStruct((ACC_ROWS, HD), jnp.float32),
            jax.ShapeDtypeStruct((ACC_ROWS, HD), jnp.float32),
        ],
        mesh=mesh,
        compiler_params=pltpu.CompilerParams(use_tc_tiling_on_sc=False),
        scratch_types=(
            [pltpu.VMEM((CHUNK,), jnp.int32)
             for _ in range(2 * NBANK * GROUP)]
            + [pltpu.VMEM((CHUNK, HD), jnp.float32)
               for _ in range(NBANK * GROUP)]
            + [
                pltpu.VMEM_SHARED((ACC_ROWS, HD), jnp.float32),
                pltpu.SemaphoreType.DMA,
                pltpu.SemaphoreType.DMA,
                pltpu.SemaphoreType.DMA,
            ]
        ),
    )
    return f(out_lo, out_hi, src2, dst2)


def _conv_body(hlo_ref, hhi_ref, w1_ref, w2_ref, b_ref, out_ref):
    hlo = hlo_ref[...]
    hhi = hhi_ref[...]
    y = lax.dot_general(hlo, w1_ref[...], (((1,), (1,)), ((), ())),
                        preferred_element_type=jnp.float32,
                         precision=lax.Precision.DEFAULT)
    y = y + lax.dot_general(hhi, w2_ref[...], (((1,), (1,)), ((), ())),
                            preferred_element_type=jnp.float32,
                         precision=lax.Precision.DEFAULT)
    out_ref[...] = jnp.maximum(y + b_ref[...], 0.0)


def _conv(h_lo, h_hi, conv_W, conv_b):
    w1 = conv_W[:, 0:HD]
    w2 = conv_W[:, HD:DIM]
    b2 = conv_b.reshape(1, DIM)
    return pl.pallas_call(
        _conv_body,
        grid=(NBA,),
        in_specs=[
            pl.BlockSpec((RA, HD), lambda i: (i, 0)),
            pl.BlockSpec((RA, HD), lambda i: (i, 0)),
            pl.BlockSpec((DIM, HD), lambda i: (0, 0)),
            pl.BlockSpec((DIM, HD), lambda i: (0, 0)),
            pl.BlockSpec((1, DIM), lambda i: (0, 0)),
        ],
        out_specs=pl.BlockSpec((RA, DIM), lambda i: (i, 0)),
        out_shape=jax.ShapeDtypeStruct((N, DIM), jnp.float32),
    )(h_lo, h_hi, w1, w2, b2)


def _set2set_body(base_ref, x_ref, bt_ref, wih_ref, whh_ref, bg_ref,
                  w1_ref, b1_ref, w2_ref, b2_ref, y_ref,
                  m_ref, ss_ref, r_ref, h_ref, c_ref, qs_ref):
    g = pl.program_id(0)
    j = g % NBC
    is_head = j == 0

    @pl.when(g == 0)
    def _init():
        qs_ref[...] = jnp.zeros_like(qs_ref)
        h_ref[...] = jnp.zeros_like(h_ref)
        c_ref[...] = jnp.zeros_like(c_ref)

    @pl.when(jnp.logical_and(is_head, g > 0))
    def _finalize():
        rv = r_ref[...] / (ss_ref[...] + 1e-16)
        qs_ref[:, 0:DIM] = h_ref[...]
        qs_ref[:, DIM:2 * DIM] = rv

    @pl.when(jnp.logical_and(is_head, g < G_GRID - 1))
    def _lstm():
        gates = lax.dot_general(qs_ref[...], wih_ref[...],
                                (((1,), (1,)), ((), ())),
                                preferred_element_type=jnp.float32,
                                precision=lax.Precision.DEFAULT)
        gates = gates + lax.dot_general(h_ref[...], whh_ref[...],
                                        (((1,), (1,)), ((), ())),
                                        preferred_element_type=jnp.float32,
                                        precision=lax.Precision.DEFAULT)
        gates = gates + bg_ref[...]
        ig = jax.nn.sigmoid(gates[:, 0:DIM])
        fg = jax.nn.sigmoid(gates[:, DIM:2 * DIM])
        gg = jnp.tanh(gates[:, 2 * DIM:3 * DIM])
        og = jax.nn.sigmoid(gates[:, 3 * DIM:4 * DIM])
        cc = fg * c_ref[...] + ig * gg
        c_ref[...] = cc
        h_ref[...] = og * jnp.tanh(cc)
        m_ref[...] = jnp.full_like(m_ref, -1e30)
        ss_ref[...] = jnp.zeros_like(ss_ref)
        r_ref[...] = jnp.zeros_like(r_ref)

    @pl.when(g < G_GRID - 1)
    def _block():
        base = base_ref[j]
        x = x_ref[...]
        bt = bt_ref[0]
        q = h_ref[pl.ds(base, WSEG), :]
        et = lax.dot_general(q, x, (((1,), (1,)), ((), ())),
                             preferred_element_type=jnp.float32,
                             precision=lax.Precision.HIGHEST)
        iot = lax.broadcasted_iota(jnp.int32, (WSEG, RC), 0) + base
        oh = iot == bt
        em = jnp.where(oh, et, -1e30)
        m_old = m_ref[pl.ds(base, WSEG)]
        m_new = jnp.maximum(m_old, jnp.max(em, axis=1, keepdims=True))
        scale = jnp.exp(m_old - m_new)
        p = jnp.exp(em - jnp.maximum(m_new, -1e20))
        ss_ref[pl.ds(base, WSEG)] = (
            ss_ref[pl.ds(base, WSEG)] * scale
            + jnp.sum(p, axis=1, keepdims=True))
        r_ref[pl.ds(base, WSEG), :] = (
            r_ref[pl.ds(base, WSEG), :] * scale
            + lax.dot_general(p, x, (((1,), (0,)), ((), ())),
                              preferred_element_type=jnp.float32,
                              precision=lax.Precision.HIGHEST))
        m_ref[pl.ds(base, WSEG)] = m_new

    @pl.when(g == G_GRID - 1)
    def _out():
        y1 = lax.dot_general(qs_ref[...], w1_ref[...],
                             (((1,), (1,)), ((), ())),
                             preferred_element_type=jnp.float32,
                             precision=lax.Precision.DEFAULT)
        y1 = jnp.maximum(y1 + b1_ref[...], 0.0)
        yt = lax.dot_general(w2_ref[...], y1, (((1,), (1,)), ((), ())),
                             preferred_element_type=jnp.float32,
                             precision=lax.Precision.DEFAULT)
        y_ref[...] = yt + b2_ref[...]


def _set2set(out2, bt3, base8, W_ih, W_hh, b_ih, b_hh, lin1_W, lin1_b,
             lin2_W, lin2_b):
    bg = (b_ih + b_hh).reshape(1, 4 * DIM)
    b1 = lin1_b.reshape(1, DIM)
    b2 = lin2_b.reshape(1, 1)
    y = pl.pallas_call(
        _set2set_body,
        grid_spec=pltpu.PrefetchScalarGridSpec(
            num_scalar_prefetch=1,
            grid=(G_GRID,),
            in_specs=[
                pl.BlockSpec((RC, DIM), lambda g, b8: (g % NBC, 0)),
                pl.BlockSpec((1, 1, RC), lambda g, b8: (g % NBC, 0, 0)),
                pl.BlockSpec((4 * DIM, 2 * DIM), lambda g, b8: (0, 0)),
                pl.BlockSpec((4 * DIM, DIM), lambda g, b8: (0, 0)),
                pl.BlockSpec((1, 4 * DIM), lambda g, b8: (0, 0)),
                pl.BlockSpec((DIM, 2 * DIM), lambda g, b8: (0, 0)),
                pl.BlockSpec((1, DIM), lambda g, b8: (0, 0)),
                pl.BlockSpec((1, DIM), lambda g, b8: (0, 0)),
                pl.BlockSpec((1, 1), lambda g, b8: (0, 0)),
            ],
            out_specs=pl.BlockSpec((1, B), lambda g, b8: (0, 0)),
            scratch_shapes=[
                pltpu.VMEM((B, 1), jnp.float32),
                pltpu.VMEM((B, 1), jnp.float32),
                pltpu.VMEM((B, DIM), jnp.float32),
                pltpu.VMEM((B, DIM), jnp.float32),
                pltpu.VMEM((B, DIM), jnp.float32),
                pltpu.VMEM((B, 2 * DIM), jnp.float32),
            ],
        ),
        out_shape=jax.ShapeDtypeStruct((1, B), jnp.float32),
    )(base8, out2, bt3, W_ih, W_hh, bg, lin1_W, b1, lin2_W, b2)
    return y.reshape(-1)


def kernel(x, edge_index, batch, lin0_W, lin0_b, conv_W, conv_b, W_ih, W_hh,
           b_ih, b_hh, lin1_W, lin1_b, lin2_W, lin2_b):
    src = edge_index[0].astype(jnp.int32)
    dst = edge_index[1].astype(jnp.int32)
    pad = E_PAD - E
    apad = jnp.arange(pad, dtype=jnp.int32)
    src2 = jnp.concatenate([src, (apad * 97) % N])
    dst2 = jnp.concatenate([dst, N + (apad % NS)])
    bt3 = batch.astype(jnp.int32).reshape(NBC, 1, RC)
    base8 = jnp.clip((bt3[:, 0, 0] // 8) * 8, 0, B - WSEG)

    out_lo, out_hi = _lin0(x, lin0_W, lin0_b)
    h_lo, h_hi = _edge_agg_sc(out_lo, out_hi, src2, dst2)
    out2 = _conv(h_lo, h_hi, conv_W, conv_b)
    return _set2set(out2, bt3, base8, W_ih, W_hh, b_ih, b_hh, lin1_W, lin1_b,
                    lin2_W, lin2_b)
